# baseline (speedup 1.0000x reference)
import jax
import jax.numpy as jnp
from jax import lax
from jax.experimental import pallas as pl
from jax.experimental.pallas import tpu as pltpu
from jax.experimental.pallas import tpu_sc as plsc

N = 10000
E = 320000
D = 128
H = 128
HH = H // 2
C = 4
G = 16

NC = 2
NS = 16
NW = NC * NS
NPAD = 10240
NACC = N
ROWS_PT = NACC // NS
EPT = E // NS
EPW = E // NW
CH = 80
NCH = EPT // CH
NCHW = EPW // CH

BLK = 512
GRID = NPAD // BLK

_mesh = lambda: plsc.VectorSubcoreMesh(core_axis_name="c", subcore_axis_name="s")


DUNR = 25


def _sc_deg_kernel(half_hbm, ones_hbm, dst2_hbm, out_hbm, idx_v, ones_v,
                   buf_v, deg_s):
    c = lax.axis_index("c")
    s = lax.axis_index("s")
    wid = s * NC + c

    pltpu.sync_copy(ones_hbm.at[pl.ds(0, CH)], ones_v)
    pltpu.sync_copy(half_hbm.at[pl.ds(s * ROWS_PT, ROWS_PT)], buf_v)
    pltpu.sync_copy(buf_v, deg_s.at[pl.ds(s * ROWS_PT, ROWS_PT)])
    plsc.subcore_barrier()

    def edge_body(it, carry):
        base = wid * (NCHW) + it * DUNR
        pltpu.sync_copy(dst2_hbm.at[pl.ds(base, DUNR)], idx_v)
        for j in range(DUNR):
            pltpu.sync_copy(ones_v, deg_s.at[idx_v.at[j]], add=True)
        return carry

    lax.fori_loop(0, NCHW // DUNR, edge_body, 0)
    plsc.subcore_barrier()
    pltpu.sync_copy(deg_s.at[pl.ds(s * ROWS_PT, ROWS_PT)], buf_v)
    pltpu.sync_copy(buf_v, out_hbm.at[c, pl.ds(s * ROWS_PT, ROWS_PT)])


def _sc_deg(half, ones, dst2):
    k = pl.kernel(
        _sc_deg_kernel,
        out_type=jax.ShapeDtypeStruct((NC, NPAD, 16), jnp.float32),
        mesh=_mesh(),
        scratch_types=[
            pltpu.VMEM((DUNR, CH), jnp.int32),
            pltpu.VMEM((CH, 16), jnp.float32),
            pltpu.VMEM((ROWS_PT, 16), jnp.float32),
            pltpu.VMEM_SHARED((NACC, 16), jnp.float32),
        ],
        compiler_params=pltpu.CompilerParams(use_tc_tiling_on_sc=False),
    )
    return k(half, ones, dst2)


NBUF = 10
KLA = 5
IB = 125


def _sc_agg_kernel(mlo_hbm, mhi_hbm, src2_hbm, dst2_hbm, out_hbm, idx_src,
                   idx_dst, *rest):
    rows = list(rest[:NBUF])
    buf_v = rest[NBUF]
    acc_s = rest[NBUF + 1]
    semi = list(rest[NBUF + 2:2 * NBUF + 2])
    semg = list(rest[2 * NBUF + 2:])
    c = lax.axis_index("c")
    s = lax.axis_index("s")

    def fire_idx(chunk, b):
        pltpu.async_copy(src2_hbm.at[chunk], idx_src.at[b], semi[b])
        pltpu.async_copy(dst2_hbm.at[chunk], idx_dst.at[b], semi[b])

    def wait_idx(b):
        pltpu.make_async_copy(src2_hbm.at[0], idx_src.at[b], semi[b]).wait()
        pltpu.make_async_copy(dst2_hbm.at[0], idx_dst.at[b], semi[b]).wait()

    def run(table):
        for k in range(ROWS_PT // IB):
            pltpu.sync_copy(table.at[pl.ds(s * ROWS_PT + k * IB, IB)], buf_v)
            pltpu.sync_copy(buf_v, acc_s.at[pl.ds(s * ROWS_PT + k * IB, IB)])
        plsc.subcore_barrier()

        base = s * NCH
        for b in range(NBUF):
            fire_idx(base + b, b)
        for j in range(KLA):
            wait_idx(j)
            pltpu.async_copy(table.at[idx_src.at[j]], rows[j], semg[j])

        def outer(it, carry):
            g = it * NBUF
            for b in range(NBUF):
                i = g + b
                bg = (b + KLA) % NBUF

                @pl.when(i + KLA < NCH)
                def _():
                    wait_idx(bg)
                    pltpu.async_copy(table.at[idx_src.at[bg]], rows[bg],
                                     semg[bg])

                pltpu.make_async_copy(table.at[idx_src.at[b]], rows[b],
                                      semg[b]).wait()
                pltpu.sync_copy(rows[b], acc_s.at[idx_dst.at[b]], add=True)

                @pl.when(i + NBUF < NCH)
                def _():
                    fire_idx(base + i + NBUF, b)
            return carry

        lax.fori_loop(0, NCH // NBUF, outer, 0)
        plsc.subcore_barrier()
        for k in range(ROWS_PT // IB):
            pltpu.sync_copy(acc_s.at[pl.ds(s * ROWS_PT + k * IB, IB)], buf_v)
            pltpu.sync_copy(buf_v,
                            out_hbm.at[c, pl.ds(s * ROWS_PT + k * IB, IB)])

    @pl.when(c == 0)
    def _lo():
        run(mlo_hbm)

    @pl.when(c == 1)
    def _hi():
        run(mhi_hbm)


def _sc_agg(mlo, mhi, src2, dst2):
    k = pl.kernel(
        _sc_agg_kernel,
        out_type=jax.ShapeDtypeStruct((NC, NPAD, HH), jnp.float32),
        mesh=_mesh(),
        scratch_types=[
            pltpu.VMEM((NBUF, CH), jnp.int32),
            pltpu.VMEM((NBUF, CH), jnp.int32),
        ] + [pltpu.VMEM((CH, HH), jnp.float32) for _ in range(NBUF)] + [
            pltpu.VMEM((IB, HH), jnp.float32),
            pltpu.VMEM_SHARED((NACC, HH), jnp.float32),
        ] + [pltpu.SemaphoreType.DMA for _ in range(2 * NBUF)],
        compiler_params=pltpu.CompilerParams(use_tc_tiling_on_sc=False),
    )
    return k(mlo, mhi, src2, dst2)


def _k1_body(x_ref, wp_ref, bp_ref, wc1_ref, deg_ref, h0_ref, mlo_ref,
             mhi_ref, dinv_ref):
    h0 = jnp.dot(x_ref[...], wp_ref[...],
                 preferred_element_type=jnp.float32) + bp_ref[...]
    dinv = lax.rsqrt(jnp.maximum(deg_ref[0] + deg_ref[1], 1.0))
    dinv_ref[...] = dinv
    h0_ref[...] = h0
    m = jnp.dot(h0, wc1_ref[...],
                preferred_element_type=jnp.float32) * dinv[:, :1]
    mlo_ref[...] = m[:, :HH]
    mhi_ref[...] = m[:, HH:]


def _tc_k1(xp, Wp, bp2, Wc1, deg):
    return pl.pallas_call(
        _k1_body,
        grid=(GRID,),
        in_specs=[
            pl.BlockSpec((BLK, D), lambda i: (i, 0)),
            pl.BlockSpec((D, H), lambda i: (0, 0)),
            pl.BlockSpec((1, H), lambda i: (0, 0)),
            pl.BlockSpec((H, H), lambda i: (0, 0)),
            pl.BlockSpec((NC, BLK, 16), lambda i: (0, i, 0)),
        ],
        out_specs=[
            pl.BlockSpec((BLK, H), lambda i: (i, 0)),
            pl.BlockSpec((BLK, HH), lambda i: (i, 0)),
            pl.BlockSpec((BLK, HH), lambda i: (i, 0)),
            pl.BlockSpec((BLK, 16), lambda i: (i, 0)),
        ],
        out_shape=[
            jax.ShapeDtypeStruct((NPAD, H), jnp.float32),
            jax.ShapeDtypeStruct((NPAD, HH), jnp.float32),
            jax.ShapeDtypeStruct((NPAD, HH), jnp.float32),
            jax.ShapeDtypeStruct((NPAD, 16), jnp.float32),
        ],
    )(xp, Wp, bp2, Wc1, deg)


def _ln_relu_res(hprev, p0, p1, dinv1, bc, g, be):
    tot = jnp.concatenate([p0, p1], axis=1) * dinv1 + bc
    h = jnp.maximum(tot + hprev, 0.0)
    mu = jnp.mean(h, axis=1, keepdims=True)
    var = jnp.mean((h - mu) * (h - mu), axis=1, keepdims=True)
    return (h - mu) / jnp.sqrt(var + 1e-5) * g + be


def _klayer_body(hprev_ref, p_ref, dinv_ref, bc_ref, g_ref, be_ref, w_ref,
                 hout_ref, mlo_ref, mhi_ref):
    dinv1 = dinv_ref[:, :1]
    hn = _ln_relu_res(hprev_ref[...], p_ref[0], p_ref[1], dinv1, bc_ref[...],
                      g_ref[...], be_ref[...])
    hout_ref[...] = hn
    m = jnp.dot(hn, w_ref[...], preferred_element_type=jnp.float32) * dinv1
    mlo_ref[...] = m[:, :HH]
    mhi_ref[...] = m[:, HH:]


def _tc_layer(hprev, p, dinv, bc2, g2, be2, Wnext):
    return pl.pallas_call(
        _klayer_body,
        grid=(GRID,),
        in_specs=[
            pl.BlockSpec((BLK, H), lambda i: (i, 0)),
            pl.BlockSpec((NC, BLK, HH), lambda i: (0, i, 0)),
            pl.BlockSpec((BLK, 16), lambda i: (i, 0)),
            pl.BlockSpec((1, H), lambda i: (0, 0)),
            pl.BlockSpec((1, H), lambda i: (0, 0)),
            pl.BlockSpec((1, H), lambda i: (0, 0)),
            pl.BlockSpec((H, H), lambda i: (0, 0)),
        ],
        out_specs=[
            pl.BlockSpec((BLK, H), lambda i: (i, 0)),
            pl.BlockSpec((BLK, HH), lambda i: (i, 0)),
            pl.BlockSpec((BLK, HH), lambda i: (i, 0)),
        ],
        out_shape=[
            jax.ShapeDtypeStruct((NPAD, H), jnp.float32),
            jax.ShapeDtypeStruct((NPAD, HH), jnp.float32),
            jax.ShapeDtypeStruct((NPAD, HH), jnp.float32),
        ],
    )(hprev, p, dinv, bc2, g2, be2, Wnext)


def _kfinal_body(hprev_ref, p_ref, dinv_ref, bc_ref, g_ref, be_ref, bat_ref,
                 wf1_ref, bf1_ref, wf2_ref, bf2_ref, out_ref):
    i = pl.program_id(0)

    @pl.when(i == 0)
    def _init():
        out_ref[...] = jnp.full((G, H), -jnp.inf, jnp.float32)

    dinv1 = dinv_ref[:, :1]
    hn = _ln_relu_res(hprev_ref[...], p_ref[0], p_ref[1], dinv1, bc_ref[...],
                      g_ref[...], be_ref[...])
    bat = bat_ref[:, :1]
    for gidx in range(G):
        mask = bat == float(gidx)
        mg = jnp.max(jnp.where(mask, hn, -jnp.inf), axis=0)
        out_ref[gidx:gidx + 1, :] = jnp.maximum(out_ref[gidx:gidx + 1, :],
                                                mg[None, :])

    @pl.when(i == GRID - 1)
    def _head():
        pooled = out_ref[...]
        z = jnp.maximum(
            jnp.dot(pooled, wf1_ref[...],
                    preferred_element_type=jnp.float32) + bf1_ref[...], 0.0)
        out_ref[...] = jnp.dot(z, wf2_ref[...],
                               preferred_element_type=jnp.float32) + bf2_ref[...]


def _tc_final(hprev, p, dinv, bc2, g2, be2, batf, Wf1, bf12, Wf2p, bf2p):
    return pl.pallas_call(
        _kfinal_body,
        grid=(GRID,),
        in_specs=[
            pl.BlockSpec((BLK, H), lambda i: (i, 0)),
            pl.BlockSpec((NC, BLK, HH), lambda i: (0, i, 0)),
            pl.BlockSpec((BLK, 16), lambda i: (i, 0)),
            pl.BlockSpec((1, H), lambda i: (0, 0)),
            pl.BlockSpec((1, H), lambda i: (0, 0)),
            pl.BlockSpec((1, H), lambda i: (0, 0)),
            pl.BlockSpec((BLK, 16), lambda i: (i, 0)),
            pl.BlockSpec((H, H), lambda i: (0, 0)),
            pl.BlockSpec((1, H), lambda i: (0, 0)),
            pl.BlockSpec((H, H), lambda i: (0, 0)),
            pl.BlockSpec((1, H), lambda i: (0, 0)),
        ],
        out_specs=pl.BlockSpec((G, H), lambda i: (0, 0)),
        out_shape=jax.ShapeDtypeStruct((G, H), jnp.float32),
    )(hprev, p, dinv, bc2, g2, be2, batf, Wf1, bf12, Wf2p, bf2p)


def kernel(x, edge_index, batch, Wp, bp, Wc1, bc1, Wc2, bc2, Wc3, bc3, g1,
           be1, g2, be2, g3, be3, Wf1, bf1, Wf2, bf2):
    src = edge_index[0]
    dst = edge_index[1]
    src2 = src.reshape(E // CH, CH)
    dst2 = dst.reshape(E // CH, CH)
    xp = jnp.pad(x, ((0, NPAD - N), (0, 0)))
    batf = jnp.broadcast_to(
        jnp.pad(batch, (0, NPAD - N), constant_values=-1).astype(
            jnp.float32)[:, None], (NPAD, 16))
    bp2 = bp[None, :]
    bc12, bc22, bc32 = bc1[None, :], bc2[None, :], bc3[None, :]
    g12, g22, g32 = g1[None, :], g2[None, :], g3[None, :]
    be12, be22, be32 = be1[None, :], be2[None, :], be3[None, :]
    bf12 = bf1[None, :]
    Wf2p = jnp.pad(Wf2, ((0, 0), (0, H - C)))
    bf2p = jnp.pad(bf2, (0, H - C))[None, :]

    halfc = jnp.full((NACC, 16), 0.5, jnp.float32)
    onesc = jnp.ones((NACC, 16), jnp.float32)
    deg2 = _sc_deg(halfc, onesc, dst2)
    h0, m1lo, m1hi, dinv = _tc_k1(xp, Wp, bp2, Wc1, deg2)
    p1 = _sc_agg(m1lo, m1hi, src2, dst2)
    h1, m2lo, m2hi = _tc_layer(h0, p1, dinv, bc12, g12, be12, Wc2)
    p2 = _sc_agg(m2lo, m2hi, src2, dst2)
    h2, m3lo, m3hi = _tc_layer(h1, p2, dinv, bc22, g22, be22, Wc3)
    p3 = _sc_agg(m3lo, m3hi, src2, dst2)
    out = _tc_final(h2, p3, dinv, bc32, g32, be32, batf, Wf1, bf12, Wf2p,
                    bf2p)
    return out[:, :C]


# speedup vs baseline: 1.0556x; 1.0556x over previous
import jax
import jax.numpy as jnp
from jax import lax
from jax.experimental import pallas as pl
from jax.experimental.pallas import tpu as pltpu
from jax.experimental.pallas import tpu_sc as plsc

N = 10000
E = 320000
D = 128
H = 128
HH = H // 2
C = 4
G = 16

NC = 2
NS = 16
NW = NC * NS
NPAD = 10240
NACC = N
ROWS_PT = NACC // NS
EPT = E // NS
EPW = E // NW
CH = 80
NCH = EPT // CH
NCHW = EPW // CH

BLK = 2048
GRID = NPAD // BLK

_mesh = lambda: plsc.VectorSubcoreMesh(core_axis_name="c", subcore_axis_name="s")


DUNR = 25


def _sc_deg_kernel(half_hbm, ones_hbm, dst2_hbm, out_hbm, idx_v, ones_v,
                   buf_v, deg_s):
    c = lax.axis_index("c")
    s = lax.axis_index("s")
    wid = s * NC + c

    pltpu.sync_copy(ones_hbm.at[pl.ds(0, CH)], ones_v)
    pltpu.sync_copy(half_hbm.at[pl.ds(s * ROWS_PT, ROWS_PT)], buf_v)
    pltpu.sync_copy(buf_v, deg_s.at[pl.ds(s * ROWS_PT, ROWS_PT)])
    plsc.subcore_barrier()

    def edge_body(it, carry):
        base = wid * (NCHW) + it * DUNR
        pltpu.sync_copy(dst2_hbm.at[pl.ds(base, DUNR)], idx_v)
        for j in range(DUNR):
            pltpu.sync_copy(ones_v, deg_s.at[idx_v.at[j]], add=True)
        return carry

    lax.fori_loop(0, NCHW // DUNR, edge_body, 0)
    plsc.subcore_barrier()
    pltpu.sync_copy(deg_s.at[pl.ds(s * ROWS_PT, ROWS_PT)], buf_v)
    pltpu.sync_copy(buf_v, out_hbm.at[c, pl.ds(s * ROWS_PT, ROWS_PT)])


def _sc_deg(half, ones, dst2):
    k = pl.kernel(
        _sc_deg_kernel,
        out_type=jax.ShapeDtypeStruct((NC, NPAD, 16), jnp.float32),
        mesh=_mesh(),
        scratch_types=[
            pltpu.VMEM((DUNR, CH), jnp.int32),
            pltpu.VMEM((CH, 16), jnp.float32),
            pltpu.VMEM((ROWS_PT, 16), jnp.float32),
            pltpu.VMEM_SHARED((NACC, 16), jnp.float32),
        ],
        compiler_params=pltpu.CompilerParams(use_tc_tiling_on_sc=False),
    )
    return k(half, ones, dst2)


NBUF = 10
KLA = 5
IB = 125


def _sc_agg_kernel(mlo_hbm, mhi_hbm, src2_hbm, dst2_hbm, out_hbm, idx_src,
                   idx_dst, *rest):
    rows = list(rest[:NBUF])
    buf_v = rest[NBUF]
    acc_s = rest[NBUF + 1]
    semi = list(rest[NBUF + 2:2 * NBUF + 2])
    semg = list(rest[2 * NBUF + 2:])
    c = lax.axis_index("c")
    s = lax.axis_index("s")

    def fire_idx(chunk, b):
        pltpu.async_copy(src2_hbm.at[chunk], idx_src.at[b], semi[b])
        pltpu.async_copy(dst2_hbm.at[chunk], idx_dst.at[b], semi[b])

    def wait_idx(b):
        pltpu.make_async_copy(src2_hbm.at[0], idx_src.at[b], semi[b]).wait()
        pltpu.make_async_copy(dst2_hbm.at[0], idx_dst.at[b], semi[b]).wait()

    def run(table):
        for k in range(ROWS_PT // IB):
            pltpu.sync_copy(table.at[pl.ds(s * ROWS_PT + k * IB, IB)], buf_v)
            pltpu.sync_copy(buf_v, acc_s.at[pl.ds(s * ROWS_PT + k * IB, IB)])
        plsc.subcore_barrier()

        base = s * NCH
        for b in range(NBUF):
            fire_idx(base + b, b)
        for j in range(KLA):
            wait_idx(j)
            pltpu.async_copy(table.at[idx_src.at[j]], rows[j], semg[j])

        def outer(it, carry):
            g = it * NBUF
            for b in range(NBUF):
                i = g + b
                bg = (b + KLA) % NBUF

                @pl.when(i + KLA < NCH)
                def _():
                    wait_idx(bg)
                    pltpu.async_copy(table.at[idx_src.at[bg]], rows[bg],
                                     semg[bg])

                pltpu.make_async_copy(table.at[idx_src.at[b]], rows[b],
                                      semg[b]).wait()
                pltpu.sync_copy(rows[b], acc_s.at[idx_dst.at[b]], add=True)

                @pl.when(i + NBUF < NCH)
                def _():
                    fire_idx(base + i + NBUF, b)
            return carry

        lax.fori_loop(0, NCH // NBUF, outer, 0)
        plsc.subcore_barrier()
        for k in range(ROWS_PT // IB):
            pltpu.sync_copy(acc_s.at[pl.ds(s * ROWS_PT + k * IB, IB)], buf_v)
            pltpu.sync_copy(buf_v,
                            out_hbm.at[c, pl.ds(s * ROWS_PT + k * IB, IB)])

    @pl.when(c == 0)
    def _lo():
        run(mlo_hbm)

    @pl.when(c == 1)
    def _hi():
        run(mhi_hbm)


def _sc_agg(mlo, mhi, src2, dst2):
    k = pl.kernel(
        _sc_agg_kernel,
        out_type=jax.ShapeDtypeStruct((NC, NPAD, HH), jnp.float32),
        mesh=_mesh(),
        scratch_types=[
            pltpu.VMEM((NBUF, CH), jnp.int32),
            pltpu.VMEM((NBUF, CH), jnp.int32),
        ] + [pltpu.VMEM((CH, HH), jnp.float32) for _ in range(NBUF)] + [
            pltpu.VMEM((IB, HH), jnp.float32),
            pltpu.VMEM_SHARED((NACC, HH), jnp.float32),
        ] + [pltpu.SemaphoreType.DMA for _ in range(2 * NBUF)],
        compiler_params=pltpu.CompilerParams(use_tc_tiling_on_sc=False),
    )
    return k(mlo, mhi, src2, dst2)


def _k1_body(x_ref, wp_ref, bp_ref, wc1_ref, deg_ref, h0_ref, mlo_ref,
             mhi_ref, dinv_ref):
    h0 = jnp.dot(x_ref[...], wp_ref[...],
                 preferred_element_type=jnp.float32) + bp_ref[...]
    dinv = lax.rsqrt(jnp.maximum(deg_ref[0] + deg_ref[1], 1.0))
    dinv_ref[...] = dinv
    h0_ref[...] = h0
    m = jnp.dot(h0, wc1_ref[...],
                preferred_element_type=jnp.float32) * dinv[:, :1]
    mlo_ref[...] = m[:, :HH]
    mhi_ref[...] = m[:, HH:]


def _tc_k1(xp, Wp, bp2, Wc1, deg):
    return pl.pallas_call(
        _k1_body,
        grid=(GRID,),
        in_specs=[
            pl.BlockSpec((BLK, D), lambda i: (i, 0)),
            pl.BlockSpec((D, H), lambda i: (0, 0)),
            pl.BlockSpec((1, H), lambda i: (0, 0)),
            pl.BlockSpec((H, H), lambda i: (0, 0)),
            pl.BlockSpec((NC, BLK, 16), lambda i: (0, i, 0)),
        ],
        out_specs=[
            pl.BlockSpec((BLK, H), lambda i: (i, 0)),
            pl.BlockSpec((BLK, HH), lambda i: (i, 0)),
            pl.BlockSpec((BLK, HH), lambda i: (i, 0)),
            pl.BlockSpec((BLK, 16), lambda i: (i, 0)),
        ],
        out_shape=[
            jax.ShapeDtypeStruct((NPAD, H), jnp.float32),
            jax.ShapeDtypeStruct((NPAD, HH), jnp.float32),
            jax.ShapeDtypeStruct((NPAD, HH), jnp.float32),
            jax.ShapeDtypeStruct((NPAD, 16), jnp.float32),
        ],
    )(xp, Wp, bp2, Wc1, deg)


def _ln_relu_res(hprev, p0, p1, dinv1, bc, g, be):
    tot = jnp.concatenate([p0, p1], axis=1) * dinv1 + bc
    h = jnp.maximum(tot + hprev, 0.0)
    mu = jnp.mean(h, axis=1, keepdims=True)
    var = jnp.mean((h - mu) * (h - mu), axis=1, keepdims=True)
    return (h - mu) / jnp.sqrt(var + 1e-5) * g + be


def _klayer_body(hprev_ref, p_ref, dinv_ref, bc_ref, g_ref, be_ref, w_ref,
                 hout_ref, mlo_ref, mhi_ref):
    dinv1 = dinv_ref[:, :1]
    hn = _ln_relu_res(hprev_ref[...], p_ref[0], p_ref[1], dinv1, bc_ref[...],
                      g_ref[...], be_ref[...])
    hout_ref[...] = hn
    m = jnp.dot(hn, w_ref[...], preferred_element_type=jnp.float32) * dinv1
    mlo_ref[...] = m[:, :HH]
    mhi_ref[...] = m[:, HH:]


def _tc_layer(hprev, p, dinv, bc2, g2, be2, Wnext):
    return pl.pallas_call(
        _klayer_body,
        grid=(GRID,),
        in_specs=[
            pl.BlockSpec((BLK, H), lambda i: (i, 0)),
            pl.BlockSpec((NC, BLK, HH), lambda i: (0, i, 0)),
            pl.BlockSpec((BLK, 16), lambda i: (i, 0)),
            pl.BlockSpec((1, H), lambda i: (0, 0)),
            pl.BlockSpec((1, H), lambda i: (0, 0)),
            pl.BlockSpec((1, H), lambda i: (0, 0)),
            pl.BlockSpec((H, H), lambda i: (0, 0)),
        ],
        out_specs=[
            pl.BlockSpec((BLK, H), lambda i: (i, 0)),
            pl.BlockSpec((BLK, HH), lambda i: (i, 0)),
            pl.BlockSpec((BLK, HH), lambda i: (i, 0)),
        ],
        out_shape=[
            jax.ShapeDtypeStruct((NPAD, H), jnp.float32),
            jax.ShapeDtypeStruct((NPAD, HH), jnp.float32),
            jax.ShapeDtypeStruct((NPAD, HH), jnp.float32),
        ],
    )(hprev, p, dinv, bc2, g2, be2, Wnext)


def _kfinal_body(hprev_ref, p_ref, dinv_ref, bc_ref, g_ref, be_ref, bat_ref,
                 wf1_ref, bf1_ref, wf2_ref, bf2_ref, out_ref):
    i = pl.program_id(0)

    @pl.when(i == 0)
    def _init():
        out_ref[...] = jnp.full((G, H), -jnp.inf, jnp.float32)

    dinv1 = dinv_ref[:, :1]
    hn = _ln_relu_res(hprev_ref[...], p_ref[0], p_ref[1], dinv1, bc_ref[...],
                      g_ref[...], be_ref[...])
    bat = bat_ref[:, :1]
    for gidx in range(G):
        mask = bat == float(gidx)
        mg = jnp.max(jnp.where(mask, hn, -jnp.inf), axis=0)
        out_ref[gidx:gidx + 1, :] = jnp.maximum(out_ref[gidx:gidx + 1, :],
                                                mg[None, :])

    @pl.when(i == GRID - 1)
    def _head():
        pooled = out_ref[...]
        z = jnp.maximum(
            jnp.dot(pooled, wf1_ref[...],
                    preferred_element_type=jnp.float32) + bf1_ref[...], 0.0)
        out_ref[...] = jnp.dot(z, wf2_ref[...],
                               preferred_element_type=jnp.float32) + bf2_ref[...]


def _tc_final(hprev, p, dinv, bc2, g2, be2, batf, Wf1, bf12, Wf2p, bf2p):
    return pl.pallas_call(
        _kfinal_body,
        grid=(GRID,),
        in_specs=[
            pl.BlockSpec((BLK, H), lambda i: (i, 0)),
            pl.BlockSpec((NC, BLK, HH), lambda i: (0, i, 0)),
            pl.BlockSpec((BLK, 16), lambda i: (i, 0)),
            pl.BlockSpec((1, H), lambda i: (0, 0)),
            pl.BlockSpec((1, H), lambda i: (0, 0)),
            pl.BlockSpec((1, H), lambda i: (0, 0)),
            pl.BlockSpec((BLK, 16), lambda i: (i, 0)),
            pl.BlockSpec((H, H), lambda i: (0, 0)),
            pl.BlockSpec((1, H), lambda i: (0, 0)),
            pl.BlockSpec((H, H), lambda i: (0, 0)),
            pl.BlockSpec((1, H), lambda i: (0, 0)),
        ],
        out_specs=pl.BlockSpec((G, H), lambda i: (0, 0)),
        out_shape=jax.ShapeDtypeStruct((G, H), jnp.float32),
    )(hprev, p, dinv, bc2, g2, be2, batf, Wf1, bf12, Wf2p, bf2p)


def kernel(x, edge_index, batch, Wp, bp, Wc1, bc1, Wc2, bc2, Wc3, bc3, g1,
           be1, g2, be2, g3, be3, Wf1, bf1, Wf2, bf2):
    src = edge_index[0]
    dst = edge_index[1]
    src2 = src.reshape(E // CH, CH)
    dst2 = dst.reshape(E // CH, CH)
    xp = jnp.pad(x, ((0, NPAD - N), (0, 0)))
    batf = jnp.broadcast_to(
        jnp.pad(batch, (0, NPAD - N), constant_values=-1).astype(
            jnp.float32)[:, None], (NPAD, 16))
    bp2 = bp[None, :]
    bc12, bc22, bc32 = bc1[None, :], bc2[None, :], bc3[None, :]
    g12, g22, g32 = g1[None, :], g2[None, :], g3[None, :]
    be12, be22, be32 = be1[None, :], be2[None, :], be3[None, :]
    bf12 = bf1[None, :]
    Wf2p = jnp.pad(Wf2, ((0, 0), (0, H - C)))
    bf2p = jnp.pad(bf2, (0, H - C))[None, :]

    halfc = jnp.full((NACC, 16), 0.5, jnp.float32)
    onesc = jnp.ones((NACC, 16), jnp.float32)
    deg2 = _sc_deg(halfc, onesc, dst2)
    h0, m1lo, m1hi, dinv = _tc_k1(xp, Wp, bp2, Wc1, deg2)
    p1 = _sc_agg(m1lo, m1hi, src2, dst2)
    h1, m2lo, m2hi = _tc_layer(h0, p1, dinv, bc12, g12, be12, Wc2)
    p2 = _sc_agg(m2lo, m2hi, src2, dst2)
    h2, m3lo, m3hi = _tc_layer(h1, p2, dinv, bc22, g22, be22, Wc3)
    p3 = _sc_agg(m3lo, m3hi, src2, dst2)
    out = _tc_final(h2, p3, dinv, bc32, g32, be32, batf, Wf1, bf12, Wf2p,
                    bf2p)
    return out[:, :C]
